# baseline (speedup 1.0000x reference)
import jax
import jax.numpy as jnp
from jax.experimental import pallas as pl
from jax.experimental.pallas import tpu as pltpu

W_BLK = 256
TOK_BLK = 256
N_FOLD = 4096 // (2 * W_BLK)


def _softmax_block(acc, bias, inv_scale):
    scaled = (acc + bias) * inv_scale
    m = jnp.max(scaled, axis=-1, keepdims=True)
    e = jnp.exp(scaled - m)
    return e / jnp.sum(e, axis=-1, keepdims=True)


def _fused_kernel(temp_ref, mk_ref, wa_ref, wb_ref, emba_ref, embb_ref,
                  embc_ref, embd_ref, b_ref, out_ref, m_acc_ref, m_bf_ref):
    t = pl.program_id(0)

    @pl.when(t < N_FOLD)
    def _fold():
        mk_a = mk_ref[:, pl.ds(t * 2 * W_BLK, W_BLK)]
        mk_b = mk_ref[:, pl.ds(t * 2 * W_BLK + W_BLK, W_BLK)]
        partial = jax.lax.dot_general(
            mk_a.astype(jnp.bfloat16), wa_ref[...].astype(jnp.bfloat16),
            dimension_numbers=(((1,), (0,)), ((), ())),
            preferred_element_type=jnp.float32,
        ) + jax.lax.dot_general(
            mk_b.astype(jnp.bfloat16), wb_ref[...].astype(jnp.bfloat16),
            dimension_numbers=(((1,), (0,)), ((), ())),
            preferred_element_type=jnp.float32,
        )

        @pl.when(t == 0)
        def _():
            m_acc_ref[...] = partial

        @pl.when(t > 0)
        def _():
            m_acc_ref[...] += partial

    @pl.when(t == N_FOLD - 1)
    def _snapshot():
        m_bf_ref[...] = m_acc_ref[...].astype(jnp.bfloat16)

    @pl.when(t >= N_FOLD)
    def _route():
        bias = jax.lax.dot_general(
            b_ref[...], mk_ref[...],
            dimension_numbers=(((1,), (1,)), ((), ())),
            preferred_element_type=jnp.float32,
        )
        temperature = jnp.maximum(jnp.exp(temp_ref[0]), 1e-4)
        inv_scale = 1.0 / (64.0 * temperature)
        for k, ref in enumerate((emba_ref, embb_ref, embc_ref, embd_ref)):
            logits = jax.lax.dot_general(
                ref[...].astype(jnp.bfloat16), m_bf_ref[...],
                dimension_numbers=(((1,), (1,)), ((), ())),
                preferred_element_type=jnp.float32,
            )
            out_ref[k * TOK_BLK:(k + 1) * TOK_BLK, :] = _softmax_block(
                logits, bias, inv_scale)


@jax.jit
def kernel(embedding, W, b, module_keys, log_temperature):
    n_tokens, d_model = embedding.shape
    n_modules = module_keys.shape[0]
    n_tok_blocks = n_tokens // (4 * TOK_BLK)

    temp = jnp.reshape(log_temperature, (1,)).astype(jnp.float32)
    b2 = jnp.reshape(b, (1, d_model))
    return pl.pallas_call(
        _fused_kernel,
        grid=(N_FOLD + n_tok_blocks,),
        in_specs=[
            pl.BlockSpec(memory_space=pltpu.SMEM),
            pl.BlockSpec((n_modules, d_model), lambda t: (0, 0)),
            pl.BlockSpec((W_BLK, d_model),
                         lambda t: (jnp.minimum(2 * t, 2 * N_FOLD - 2), 0)),
            pl.BlockSpec((W_BLK, d_model),
                         lambda t: (jnp.minimum(2 * t + 1, 2 * N_FOLD - 1), 0)),
            pl.BlockSpec((TOK_BLK, d_model),
                         lambda t: (jnp.maximum(4 * (t - N_FOLD), 0), 0)),
            pl.BlockSpec((TOK_BLK, d_model),
                         lambda t: (jnp.maximum(4 * (t - N_FOLD) + 1, 1), 0)),
            pl.BlockSpec((TOK_BLK, d_model),
                         lambda t: (jnp.maximum(4 * (t - N_FOLD) + 2, 2), 0)),
            pl.BlockSpec((TOK_BLK, d_model),
                         lambda t: (jnp.maximum(4 * (t - N_FOLD) + 3, 3), 0)),
            pl.BlockSpec((1, d_model), lambda t: (0, 0)),
        ],
        out_specs=pl.BlockSpec((4 * TOK_BLK, n_modules),
                               lambda t: (jnp.maximum(t - N_FOLD, 0), 0)),
        out_shape=jax.ShapeDtypeStruct((n_tokens, n_modules), jnp.float32),
        scratch_shapes=[
            pltpu.VMEM((n_modules, d_model), jnp.float32),
            pltpu.VMEM((n_modules, d_model), jnp.bfloat16),
        ],
        compiler_params=pltpu.CompilerParams(
            dimension_semantics=("arbitrary",)),
    )(temp, module_keys, W, W, embedding, embedding, embedding, embedding,
      b2)


# speedup vs baseline: 1.0044x; 1.0044x over previous
import jax
import jax.numpy as jnp
from jax.experimental import pallas as pl
from jax.experimental.pallas import tpu as pltpu

W_BLK = 128
TOK_BLK = 128
N_FOLD = 4096 // (4 * W_BLK)


def _softmax_block(acc, bias, inv_scale):
    scaled = (acc + bias) * inv_scale
    m = jnp.max(scaled, axis=-1, keepdims=True)
    e = jnp.exp(scaled - m)
    return e / jnp.sum(e, axis=-1, keepdims=True)


def _fused_kernel(temp_ref, mk_ref, wa_ref, wb_ref, wc_ref, wd_ref,
                  e0_ref, e1_ref, e2_ref, e3_ref, e4_ref, e5_ref, e6_ref,
                  e7_ref, b_ref, out_ref, m_acc_ref, m_bf_ref):
    t = pl.program_id(0)

    @pl.when(t < N_FOLD)
    def _fold():
        partial = jnp.zeros((mk_ref.shape[0], mk_ref.shape[1]), jnp.float32)
        for s, wref in enumerate((wa_ref, wb_ref, wc_ref, wd_ref)):
            mk_s = mk_ref[:, pl.ds((4 * t + s) * W_BLK, W_BLK)]
            partial += jax.lax.dot_general(
                mk_s.astype(jnp.bfloat16), wref[...].astype(jnp.bfloat16),
                dimension_numbers=(((1,), (0,)), ((), ())),
                preferred_element_type=jnp.float32,
            )

        @pl.when(t == 0)
        def _():
            m_acc_ref[...] = partial

        @pl.when(t > 0)
        def _():
            m_acc_ref[...] += partial

    @pl.when(t == N_FOLD - 1)
    def _snapshot():
        m_bf_ref[...] = m_acc_ref[...].astype(jnp.bfloat16)

    @pl.when(t >= N_FOLD)
    def _route():
        bias = jax.lax.dot_general(
            b_ref[...], mk_ref[...],
            dimension_numbers=(((1,), (1,)), ((), ())),
            preferred_element_type=jnp.float32,
        )
        temperature = jnp.maximum(jnp.exp(temp_ref[0]), 1e-4)
        inv_scale = 1.0 / (64.0 * temperature)
        for k, eref in enumerate((e0_ref, e1_ref, e2_ref, e3_ref, e4_ref,
                                  e5_ref, e6_ref, e7_ref)):
            logits = jax.lax.dot_general(
                eref[...].astype(jnp.bfloat16), m_bf_ref[...],
                dimension_numbers=(((1,), (1,)), ((), ())),
                preferred_element_type=jnp.float32,
            )
            out_ref[k * TOK_BLK:(k + 1) * TOK_BLK, :] = _softmax_block(
                logits, bias, inv_scale)


@jax.jit
def kernel(embedding, W, b, module_keys, log_temperature):
    n_tokens, d_model = embedding.shape
    n_modules = module_keys.shape[0]
    n_tok_blocks = n_tokens // (8 * TOK_BLK)

    temp = jnp.reshape(log_temperature, (1,)).astype(jnp.float32)
    b2 = jnp.reshape(b, (1, d_model))
    return pl.pallas_call(
        _fused_kernel,
        grid=(N_FOLD + n_tok_blocks,),
        in_specs=[
            pl.BlockSpec(memory_space=pltpu.SMEM),
            pl.BlockSpec((n_modules, d_model), lambda t: (0, 0)),
            *[pl.BlockSpec(
                  (W_BLK, d_model),
                  lambda t, s=s: (jnp.minimum(4 * t + s, 4 * N_FOLD - 4 + s),
                                  0))
              for s in range(4)],
            *[pl.BlockSpec(
                  (TOK_BLK, d_model),
                  lambda t, k=k: (jnp.maximum(8 * (t - N_FOLD) + k, k), 0))
              for k in range(8)],
            pl.BlockSpec((1, d_model), lambda t: (0, 0)),
        ],
        out_specs=pl.BlockSpec((8 * TOK_BLK, n_modules),
                               lambda t: (jnp.maximum(t - N_FOLD, 0), 0)),
        out_shape=jax.ShapeDtypeStruct((n_tokens, n_modules), jnp.float32),
        scratch_shapes=[
            pltpu.VMEM((n_modules, d_model), jnp.float32),
            pltpu.VMEM((n_modules, d_model), jnp.bfloat16),
        ],
        compiler_params=pltpu.CompilerParams(
            dimension_semantics=("arbitrary",)),
    )(temp, module_keys, W, W, W, W, embedding, embedding, embedding,
      embedding, embedding, embedding, embedding, embedding, b2)
